# baseline (speedup 1.0000x reference)
import jax
import jax.numpy as jnp
from jax.experimental import pallas as pl

_NSEG = 64
_D = 45
_P = 8


def _mlp(x, w0, b0, w1, b1, w2, b2, w3, b3):
    h = jnp.maximum(jnp.dot(x, w0, preferred_element_type=jnp.float32) + b0, 0.0)
    h = jnp.maximum(jnp.dot(h, w1, preferred_element_type=jnp.float32) + b1, 0.0)
    h = jnp.maximum(jnp.dot(h, w2, preferred_element_type=jnp.float32) + b2, 0.0)
    return jnp.dot(h, w3, preferred_element_type=jnp.float32) + b3


def _pair_kernel(c_ref, x_ref, ids_ref, w0_ref, b0_ref, w1_ref, b1_ref,
                 w2_ref, b2_ref, w3_ref, b3_ref, out_ref):
    i = pl.program_id(0)
    args = (w0_ref[...], b0_ref[...], w1_ref[...], b1_ref[...],
            w2_ref[...], b2_ref[...], w3_ref[...], b3_ref[...])
    sc = _mlp(c_ref[0], *args)
    sx = _mlp(x_ref[0], *args)
    d = sc - sx
    ids = ids_ref[0]
    tile_p = d.shape[0]
    seg = jax.lax.broadcasted_iota(jnp.int32, (tile_p, _NSEG), 1)
    acc = jnp.zeros((tile_p, _NSEG), jnp.float32)
    for p in range(_P):
        acc += jnp.where(ids[:, p:p + 1] == seg, d[:, p:p + 1], 0.0)
    partial = jnp.sum(acc, axis=0, keepdims=True)

    @pl.when(i == 0)
    def _():
        out_ref[...] = jnp.zeros_like(out_ref)

    out_ref[...] += partial


def _run_pair(cx3, off, x3, ids3, ws):
    nsteps, tile_p, wide = x3.shape
    w0, b0, w1, b1, w2, b2, w3, b3 = ws
    wspec = lambda a: pl.BlockSpec(a.shape, lambda i: (0,) * a.ndim)
    return pl.pallas_call(
        _pair_kernel,
        grid=(nsteps,),
        in_specs=[
            pl.BlockSpec((1, tile_p, wide), lambda i, off=off: (off + i, 0, 0)),
            pl.BlockSpec((1, tile_p, wide), lambda i: (i, 0, 0)),
            pl.BlockSpec((1, tile_p, _P), lambda i: (i, 0, 0)),
            wspec(w0), wspec(b0), wspec(w1), wspec(b1),
            wspec(w2), wspec(b2), wspec(w3), wspec(b3),
        ],
        out_specs=pl.BlockSpec((1, _NSEG), lambda i: (0, 0)),
        out_shape=jax.ShapeDtypeStruct((1, _NSEG), jnp.float32),
    )(cx3, x3, ids3, w0, b0, w1, b1, w2, b2, w3, b3)


def kernel(protein_conv_out, ligand_conv_out, complex_conv_out,
           protein_segment_ids, ligand_segment_ids,
           W0, b0, W1, b1, W2, b2, W3, b3):
    v1 = protein_conv_out.shape[0]
    v2 = ligand_conv_out.shape[0]
    tile = 2000
    tile_p = tile // _P
    wide = _P * _D
    nsplit = 5
    chunk = v1 // nsplit
    eye = jnp.eye(_P, dtype=jnp.float32)
    ws = (jnp.kron(eye, W0), jnp.tile(b0, _P).reshape(1, -1),
          jnp.kron(eye, W1), jnp.tile(b1, _P).reshape(1, -1),
          jnp.kron(eye, W2), jnp.tile(b2, _P).reshape(1, -1),
          jnp.kron(eye, W3), jnp.tile(b3, _P).reshape(1, -1))
    parts = []
    for k in range(nsplit):
        r0, r1 = k * chunk, (k + 1) * chunk
        cxk = complex_conv_out[r0:r1].reshape(chunk // tile, tile_p, wide)
        pk = protein_conv_out[r0:r1].reshape(chunk // tile, tile_p, wide)
        idsk = protein_segment_ids[r0:r1].reshape(chunk // tile, tile_p, _P)
        parts.append(_run_pair(cxk, 0, pk, idsk, ws))
    cxb = complex_conv_out[v1:].reshape(v2 // tile, tile_p, wide)
    lb = ligand_conv_out.reshape(v2 // tile, tile_p, wide)
    idsb = ligand_segment_ids.reshape(v2 // tile, tile_p, _P)
    parts.append(_run_pair(cxb, 0, lb, idsb, ws))
    total = parts[0]
    for p in parts[1:]:
        total = total + p
    return total.reshape(_NSEG, 1)


# speedup vs baseline: 7.1190x; 7.1190x over previous
import functools

import jax
import jax.numpy as jnp
from jax.experimental import pallas as pl
from jax.experimental.pallas import tpu as pltpu

_NSEG = 64
_D = 45
_V1 = 100000
_V2 = 10000
_TILE_A = 9088
_NA = 11
_ALN_A = _NA * _TILE_A
_TILE_B = 9984
_W = 128


def _mlp_t(x, w0t, b0, w1t, b1, w2t, b2, w3t, b3):
    h = jnp.maximum(jnp.dot(w0t, x, preferred_element_type=jnp.float32) + b0, 0.0)
    h = jnp.maximum(jnp.dot(w1t, h, preferred_element_type=jnp.float32) + b1, 0.0)
    h = jnp.maximum(jnp.dot(w2t, h, preferred_element_type=jnp.float32) + b2, 0.0)
    return jnp.dot(w3t, h, preferred_element_type=jnp.float32) + b3


def _kernel(cx_ref, pt_ref, lg_ref, cxb_ref, pid_ref, lid_ref,
            tca_ref, tp_ref, tid_a_ref, tcb_ref, tl_ref, tid_b_ref,
            w0t_ref, b0_ref, w1t_ref, b1_ref, w2t_ref, b2_ref,
            w3t_ref, b3_ref, out_ref,
            cbuf, xbuf, idbuf, acc, sem_c, sem_x, sem_i):
    i = pl.program_id(0)
    nsteps = _NA + 2

    def start(step, slot):
        @pl.when(step < _NA)
        def _():
            pltpu.make_async_copy(
                cx_ref.at[:, pl.ds(step * _TILE_A, _TILE_A)],
                cbuf.at[slot, :, pl.ds(0, _TILE_A)], sem_c.at[slot]).start()
            pltpu.make_async_copy(
                pt_ref.at[:, pl.ds(step * _TILE_A, _TILE_A)],
                xbuf.at[slot, :, pl.ds(0, _TILE_A)], sem_x.at[slot]).start()
            pltpu.make_async_copy(
                pid_ref.at[:, pl.ds(step * _TILE_A, _TILE_A)],
                idbuf.at[slot, :, pl.ds(0, _TILE_A)], sem_i.at[slot]).start()

        @pl.when(step == _NA)
        def _():
            pltpu.make_async_copy(
                cxb_ref.at[:, pl.ds(0, _TILE_B)],
                cbuf.at[slot, :, pl.ds(0, _TILE_B)], sem_c.at[slot]).start()
            pltpu.make_async_copy(
                lg_ref.at[:, pl.ds(0, _TILE_B)],
                xbuf.at[slot, :, pl.ds(0, _TILE_B)], sem_x.at[slot]).start()
            pltpu.make_async_copy(
                lid_ref.at[:, pl.ds(0, _TILE_B)],
                idbuf.at[slot, :, pl.ds(0, _TILE_B)], sem_i.at[slot]).start()

    def wait(step, slot):
        @pl.when(step < _NA)
        def _():
            pltpu.make_async_copy(
                cx_ref.at[:, pl.ds(0, _TILE_A)],
                cbuf.at[slot, :, pl.ds(0, _TILE_A)], sem_c.at[slot]).wait()
            pltpu.make_async_copy(
                pt_ref.at[:, pl.ds(0, _TILE_A)],
                xbuf.at[slot, :, pl.ds(0, _TILE_A)], sem_x.at[slot]).wait()
            pltpu.make_async_copy(
                pid_ref.at[:, pl.ds(0, _TILE_A)],
                idbuf.at[slot, :, pl.ds(0, _TILE_A)], sem_i.at[slot]).wait()

        @pl.when(step == _NA)
        def _():
            pltpu.make_async_copy(
                cxb_ref.at[:, pl.ds(0, _TILE_B)],
                cbuf.at[slot, :, pl.ds(0, _TILE_B)], sem_c.at[slot]).wait()
            pltpu.make_async_copy(
                lg_ref.at[:, pl.ds(0, _TILE_B)],
                xbuf.at[slot, :, pl.ds(0, _TILE_B)], sem_x.at[slot]).wait()
            pltpu.make_async_copy(
                lid_ref.at[:, pl.ds(0, _TILE_B)],
                idbuf.at[slot, :, pl.ds(0, _TILE_B)], sem_i.at[slot]).wait()

    slot = jax.lax.rem(i, 2)

    @pl.when(i == 0)
    def _():
        acc[...] = jnp.zeros_like(acc)
        start(0, 0)

    @pl.when(i + 1 < nsteps)
    def _():
        start(i + 1, jax.lax.rem(i + 1, 2))

    wait(i, slot)

    args = (w0t_ref[...], b0_ref[...], w1t_ref[...], b1_ref[...],
            w2t_ref[...], b2_ref[...], w3t_ref[...], b3_ref[...])

    def accum(c, x, ids, width, valid_from=None):
        d = _mlp_t(c, *args) - _mlp_t(x, *args)
        seg = jax.lax.broadcasted_iota(jnp.int32, (_NSEG, width), 0)
        mask = ids == seg
        if valid_from is not None:
            lane = jax.lax.broadcasted_iota(jnp.int32, (_NSEG, width), 1)
            mask = jnp.logical_and(mask, lane >= valid_from)
        contrib = jnp.where(mask, jnp.broadcast_to(d, (_NSEG, width)), 0.0)
        acc[:, pl.ds(0, width)] += contrib

    @pl.when(i < _NA)
    def _():
        accum(cbuf[slot, :, pl.ds(0, _TILE_A)],
              xbuf[slot, :, pl.ds(0, _TILE_A)],
              idbuf[slot, :, pl.ds(0, _TILE_A)], _TILE_A)

    @pl.when(i == _NA)
    def _():
        accum(cbuf[slot, :, pl.ds(0, _TILE_B)],
              xbuf[slot, :, pl.ds(0, _TILE_B)],
              idbuf[slot, :, pl.ds(0, _TILE_B)], _TILE_B)

    @pl.when(i == nsteps - 1)
    def _():
        accum(tca_ref[...], tp_ref[...], tid_a_ref[...], _W,
              valid_from=_W - (_V1 % _W))
        accum(tcb_ref[...], tl_ref[...], tid_b_ref[...], _W,
              valid_from=_W - (_V2 % _W))
        out_ref[...] = jnp.sum(acc[...], axis=1, keepdims=True)


def kernel(protein_conv_out, ligand_conv_out, complex_conv_out,
           protein_segment_ids, ligand_segment_ids,
           W0, b0, W1, b1, W2, b2, W3, b3):
    v1 = protein_conv_out.shape[0]
    v2 = ligand_conv_out.shape[0]
    cxT = complex_conv_out.T
    ptT = protein_conv_out.T
    lgT = ligand_conv_out.T
    cxbT = cxT[:, v1:v1 + _TILE_B]
    pid2 = protein_segment_ids.reshape(1, v1)
    lid2 = ligand_segment_ids.reshape(1, v2)
    tca = cxT[:, v1 - _W:v1]
    tp = ptT[:, v1 - _W:]
    tid_a = pid2[:, v1 - _W:]
    tcb = cxT[:, v1 + v2 - _W:]
    tl = lgT[:, v2 - _W:]
    tid_b = lid2[:, v2 - _W:]
    hbm = pl.BlockSpec(memory_space=pl.ANY)
    vmem = lambda a: pl.BlockSpec(a.shape, lambda i: (0,) * a.ndim)
    ws = (W0.T, b0.reshape(-1, 1), W1.T, b1.reshape(-1, 1),
          W2.T, b2.reshape(-1, 1), W3.T, b3.reshape(-1, 1))
    tails = (tca, tp, tid_a, tcb, tl, tid_b)
    out = pl.pallas_call(
        _kernel,
        grid=(_NA + 2,),
        in_specs=[hbm] * 6 + [vmem(t) for t in tails] + [vmem(w) for w in ws],
        out_specs=pl.BlockSpec((_NSEG, 1), lambda i: (0, 0)),
        out_shape=jax.ShapeDtypeStruct((_NSEG, 1), jnp.float32),
        scratch_shapes=[
            pltpu.VMEM((2, _D, _TILE_B), jnp.float32),
            pltpu.VMEM((2, _D, _TILE_B), jnp.float32),
            pltpu.VMEM((2, 1, _TILE_B), jnp.int32),
            pltpu.VMEM((_NSEG, _TILE_B), jnp.float32),
            pltpu.SemaphoreType.DMA((2,)),
            pltpu.SemaphoreType.DMA((2,)),
            pltpu.SemaphoreType.DMA((2,)),
        ],
        compiler_params=pltpu.CompilerParams(
            dimension_semantics=("arbitrary",)),
    )(cxT, ptT, lgT, cxbT, pid2, lid2, *tails, *ws)
    return out


# speedup vs baseline: 7.3244x; 1.0289x over previous
import functools

import jax
import jax.numpy as jnp
from jax.experimental import pallas as pl
from jax.experimental.pallas import tpu as pltpu

_NSEG = 64
_D = 45
_V1 = 100000
_V2 = 10000
_TILE_A = 9088
_NA = 11
_ALN_A = _NA * _TILE_A
_TILE_B = 9984
_W = 128


def _mlp_t(x, w0t, b0, w1t, b1, w2t, b2, w3t, b3):
    h = jnp.maximum(jnp.dot(w0t, x, preferred_element_type=jnp.float32) + b0, 0.0)
    h = jnp.maximum(jnp.dot(w1t, h, preferred_element_type=jnp.float32) + b1, 0.0)
    h = jnp.maximum(jnp.dot(w2t, h, preferred_element_type=jnp.float32) + b2, 0.0)
    return jnp.dot(w3t, h, preferred_element_type=jnp.float32) + b3


def _kernel(cx_ref, pt_ref, lg_ref, cxb_ref, pid_ref, lid_ref,
            tca_ref, tp_ref, tid_a_ref, tcb_ref, tl_ref, tid_b_ref,
            w0t_ref, b0_ref, w1t_ref, b1_ref, w2t_ref, b2_ref,
            w3t_ref, b3_ref, out_ref,
            cbuf, idbuf, acc, sem_c, sem_x, sem_i):
    i = pl.program_id(0)
    nsteps = _NA + 2

    def start(step, slot):
        @pl.when(step < _NA)
        def _():
            pltpu.make_async_copy(
                cx_ref.at[:, pl.ds(step * _TILE_A, _TILE_A)],
                cbuf.at[slot, :, pl.ds(0, _TILE_A)], sem_c.at[slot]).start()
            pltpu.make_async_copy(
                pt_ref.at[:, pl.ds(step * _TILE_A, _TILE_A)],
                cbuf.at[slot, :, pl.ds(_TILE_A, _TILE_A)], sem_x.at[slot]).start()
            pltpu.make_async_copy(
                pid_ref.at[:, pl.ds(step * _TILE_A, _TILE_A)],
                idbuf.at[slot, :, pl.ds(0, _TILE_A)], sem_i.at[slot]).start()

        @pl.when(step == _NA)
        def _():
            pltpu.make_async_copy(
                cxb_ref.at[:, pl.ds(0, _TILE_B)],
                cbuf.at[slot, :, pl.ds(0, _TILE_B)], sem_c.at[slot]).start()
            pltpu.make_async_copy(
                lg_ref.at[:, pl.ds(0, _TILE_B)],
                cbuf.at[slot, :, pl.ds(_TILE_B, _TILE_B)], sem_x.at[slot]).start()
            pltpu.make_async_copy(
                lid_ref.at[:, pl.ds(0, _TILE_B)],
                idbuf.at[slot, :, pl.ds(0, _TILE_B)], sem_i.at[slot]).start()

    def wait(step, slot):
        @pl.when(step < _NA)
        def _():
            pltpu.make_async_copy(
                cx_ref.at[:, pl.ds(0, _TILE_A)],
                cbuf.at[slot, :, pl.ds(0, _TILE_A)], sem_c.at[slot]).wait()
            pltpu.make_async_copy(
                pt_ref.at[:, pl.ds(0, _TILE_A)],
                cbuf.at[slot, :, pl.ds(_TILE_A, _TILE_A)], sem_x.at[slot]).wait()
            pltpu.make_async_copy(
                pid_ref.at[:, pl.ds(0, _TILE_A)],
                idbuf.at[slot, :, pl.ds(0, _TILE_A)], sem_i.at[slot]).wait()

        @pl.when(step == _NA)
        def _():
            pltpu.make_async_copy(
                cxb_ref.at[:, pl.ds(0, _TILE_B)],
                cbuf.at[slot, :, pl.ds(0, _TILE_B)], sem_c.at[slot]).wait()
            pltpu.make_async_copy(
                lg_ref.at[:, pl.ds(0, _TILE_B)],
                cbuf.at[slot, :, pl.ds(_TILE_B, _TILE_B)], sem_x.at[slot]).wait()
            pltpu.make_async_copy(
                lid_ref.at[:, pl.ds(0, _TILE_B)],
                idbuf.at[slot, :, pl.ds(0, _TILE_B)], sem_i.at[slot]).wait()

    slot = jax.lax.rem(i, 2)

    @pl.when(i == 0)
    def _():
        acc[...] = jnp.zeros_like(acc)
        start(0, 0)

    @pl.when(i + 1 < nsteps)
    def _():
        start(i + 1, jax.lax.rem(i + 1, 2))

    wait(i, slot)

    args = (w0t_ref[...], b0_ref[...], w1t_ref[...], b1_ref[...],
            w2t_ref[...], b2_ref[...], w3t_ref[...], b3_ref[...])

    def accum_d(d, ids, width, valid_from=None):
        seg = jax.lax.broadcasted_iota(jnp.int32, (_NSEG, width), 0)
        mask = ids == seg
        if valid_from is not None:
            lane = jax.lax.broadcasted_iota(jnp.int32, (_NSEG, width), 1)
            mask = jnp.logical_and(mask, lane >= valid_from)
        contrib = jnp.where(mask, jnp.broadcast_to(d, (_NSEG, width)), 0.0)
        acc[:, pl.ds(0, width)] += contrib

    def accum_pair(z, ids, width):
        d = jax.lax.slice(z, (0, 0), (1, width)) - \
            jax.lax.slice(z, (0, width), (1, 2 * width))
        accum_d(d, ids, width)

    def accum(c, x, ids, width, valid_from=None):
        d = _mlp_t(c, *args) - _mlp_t(x, *args)
        accum_d(d, ids, width, valid_from)

    @pl.when(i < _NA)
    def _():
        z = _mlp_t(cbuf[slot, :, pl.ds(0, 2 * _TILE_A)], *args)
        accum_pair(z, idbuf[slot, :, pl.ds(0, _TILE_A)], _TILE_A)

    @pl.when(i == _NA)
    def _():
        z = _mlp_t(cbuf[slot, :, pl.ds(0, 2 * _TILE_B)], *args)
        accum_pair(z, idbuf[slot, :, pl.ds(0, _TILE_B)], _TILE_B)

    @pl.when(i == nsteps - 1)
    def _():
        accum(tca_ref[...], tp_ref[...], tid_a_ref[...], _W,
              valid_from=_W - (_V1 % _W))
        accum(tcb_ref[...], tl_ref[...], tid_b_ref[...], _W,
              valid_from=_W - (_V2 % _W))
        out_ref[...] = jnp.sum(acc[...], axis=1, keepdims=True)


def kernel(protein_conv_out, ligand_conv_out, complex_conv_out,
           protein_segment_ids, ligand_segment_ids,
           W0, b0, W1, b1, W2, b2, W3, b3):
    v1 = protein_conv_out.shape[0]
    v2 = ligand_conv_out.shape[0]
    cxT = complex_conv_out.T
    ptT = protein_conv_out.T
    lgT = ligand_conv_out.T
    cxbT = cxT[:, v1:v1 + _TILE_B]
    pid2 = protein_segment_ids.reshape(1, v1)
    lid2 = ligand_segment_ids.reshape(1, v2)
    tca = cxT[:, v1 - _W:v1]
    tp = ptT[:, v1 - _W:]
    tid_a = pid2[:, v1 - _W:]
    tcb = cxT[:, v1 + v2 - _W:]
    tl = lgT[:, v2 - _W:]
    tid_b = lid2[:, v2 - _W:]
    hbm = pl.BlockSpec(memory_space=pl.ANY)
    vmem = lambda a: pl.BlockSpec(a.shape, lambda i: (0,) * a.ndim)
    ws = (W0.T, b0.reshape(-1, 1), W1.T, b1.reshape(-1, 1),
          W2.T, b2.reshape(-1, 1), W3.T, b3.reshape(-1, 1))
    tails = (tca, tp, tid_a, tcb, tl, tid_b)
    out = pl.pallas_call(
        _kernel,
        grid=(_NA + 2,),
        in_specs=[hbm] * 6 + [vmem(t) for t in tails] + [vmem(w) for w in ws],
        out_specs=pl.BlockSpec((_NSEG, 1), lambda i: (0, 0)),
        out_shape=jax.ShapeDtypeStruct((_NSEG, 1), jnp.float32),
        scratch_shapes=[
            pltpu.VMEM((2, _D, 2 * _TILE_B), jnp.float32),
            pltpu.VMEM((2, 1, _TILE_B), jnp.int32),
            pltpu.VMEM((_NSEG, _TILE_B), jnp.float32),
            pltpu.SemaphoreType.DMA((2,)),
            pltpu.SemaphoreType.DMA((2,)),
            pltpu.SemaphoreType.DMA((2,)),
        ],
        compiler_params=pltpu.CompilerParams(
            dimension_semantics=("arbitrary",)),
    )(cxT, ptT, lgT, cxbT, pid2, lid2, *tails, *ws)
    return out
